# baseline (speedup 1.0000x reference)
import functools

import jax
import jax.numpy as jnp
from jax import lax
from jax.experimental import pallas as pl
from jax.experimental.pallas import tpu as pltpu
from jax.experimental.pallas import tpu_sc as plsc

N = 10000
E = 320000
NC, NS = 2, 16
NW = NC * NS
CHUNK = 128
EPT = 10240
NCHUNKS = EPT // CHUNK
EPAD = NW * EPT
NPAD = 10240
RPT = NPAD // NS

_MESH = plsc.VectorSubcoreMesh(core_axis_name="c", subcore_axis_name="s")


def _make_edge_scatter(D, staged=False, tc_tiling=True):
    scratch = [
        pltpu.VMEM((NCHUNKS, CHUNK), jnp.int32),
        [pltpu.VMEM((CHUNK,), jnp.int32) for _ in range(2)],
        [pltpu.VMEM((CHUNK, D), jnp.float32) for _ in range(2)],
        pltpu.VMEM_SHARED((NPAD, D), jnp.float32),
        pltpu.SemaphoreType.DMA((2,)),
        pltpu.SemaphoreType.DMA((2,)),
    ]
    if staged:
        scratch.append(pltpu.VMEM_SHARED((NPAD, D), jnp.float32))

    @functools.partial(
        pl.kernel,
        out_type=jax.ShapeDtypeStruct((NC, NPAD, D), jnp.float32),
        mesh=_MESH,
        scratch_types=scratch,
        compiler_params=pltpu.CompilerParams(use_tc_tiling_on_sc=tc_tiling),
    )
    def k(y_hbm, src_hbm, dst_hbm, zeros_hbm, out_hbm,
          didx, sidx, rows, acc, sem_g, sem_i, *maybe_table):
        c = lax.axis_index("c")
        s = lax.axis_index("s")
        wid = c * NS + s
        pltpu.sync_copy(zeros_hbm.at[pl.ds(s * RPT, RPT)],
                        acc.at[pl.ds(s * RPT, RPT)])
        if staged:
            table = maybe_table[0]
            pltpu.sync_copy(y_hbm.at[pl.ds(s * RPT, RPT)],
                            table.at[pl.ds(s * RPT, RPT)])
        else:
            table = y_hbm
        pltpu.sync_copy(dst_hbm.at[wid], didx)
        pltpu.sync_copy(src_hbm.at[wid, 0], sidx[0])
        pltpu.sync_copy(src_hbm.at[wid, 1], sidx[1])
        plsc.subcore_barrier()

        def start_sidx(v, b):
            pltpu.async_copy(src_hbm.at[wid, v], sidx[b], sem_i.at[b])

        def wait_sidx(v, b):
            pltpu.make_async_copy(src_hbm.at[wid, v], sidx[b],
                                  sem_i.at[b]).wait()

        def start_gather(b):
            pltpu.async_copy(table.at[sidx[b]], rows[b], sem_g.at[b])

        def wait_gather(b):
            pltpu.make_async_copy(table.at[sidx[b]], rows[b],
                                  sem_g.at[b]).wait()

        start_gather(0)
        start_gather(1)

        @pl.loop(0, NCHUNKS, step=2)
        def _(kk):
            for b in range(2):
                v = kk + b
                wait_gather(b)

                @pl.when(v + 2 < NCHUNKS)
                def _():
                    start_sidx(v + 2, b)

                pltpu.sync_copy(rows[b], acc.at[didx.at[v]], add=True)

                @pl.when(v + 2 < NCHUNKS)
                def _():
                    wait_sidx(v + 2, b)
                    start_gather(b)

        plsc.subcore_barrier()
        pltpu.sync_copy(acc.at[pl.ds(s * RPT, RPT)],
                        out_hbm.at[c, pl.ds(s * RPT, RPT)])

    return k


@functools.partial(
    pl.kernel,
    out_type=jax.ShapeDtypeStruct((NC, NPAD, 16), jnp.float32),
    mesh=_MESH,
    scratch_types=[
        pltpu.VMEM((NCHUNKS, CHUNK), jnp.int32),
        pltpu.VMEM((CHUNK, 16), jnp.float32),
        pltpu.VMEM_SHARED((NPAD, 16), jnp.float32),
        pltpu.SemaphoreType.DMA((1,)),
    ],
    compiler_params=pltpu.CompilerParams(use_tc_tiling_on_sc=False),
)
def _degree_kernel(dst_hbm, ones_hbm, zeros_hbm, out_hbm, didx, ones_v, acc,
                   sem):
    c = lax.axis_index("c")
    s = lax.axis_index("s")
    wid = c * NS + s
    pltpu.sync_copy(zeros_hbm.at[pl.ds(s * RPT, RPT)],
                    acc.at[pl.ds(s * RPT, RPT)])
    pltpu.sync_copy(ones_hbm, ones_v)
    pltpu.sync_copy(dst_hbm.at[wid], didx)
    plsc.subcore_barrier()

    @pl.loop(0, NCHUNKS, step=8)
    def _(kk):
        for j in range(8):
            pltpu.async_copy(ones_v, acc.at[didx.at[kk + j]], sem.at[0],
                             add=True)
        for j in range(8):
            pltpu.make_async_copy(ones_v, acc.at[didx.at[kk + j]],
                                  sem.at[0]).wait()

    plsc.subcore_barrier()
    pltpu.sync_copy(acc.at[pl.ds(s * RPT, RPT)],
                    out_hbm.at[c, pl.ds(s * RPT, RPT)])


def _mm_body(x_ref, w_ref, o_ref):
    o_ref[...] = jnp.dot(x_ref[...], w_ref[...],
                         preferred_element_type=jnp.float32)


def _dinv_scale_body(degp_ref, xw_ref, dinv_ref, y_ref):
    d = degp_ref[...]
    deg = d[0, :, 0:1] + d[1, :, 0:1] + 1.0
    dinv = lax.rsqrt(deg)
    dinv_ref[...] = dinv
    y_ref[...] = xw_ref[...] * dinv


def _mid_body(s_ref, y1_ref, dinv_ref, b1_ref, w2_ref, y2_ref):
    sp = s_ref[...]
    dinv = dinv_ref[...]
    h = jnp.maximum((sp[0] + sp[1] + y1_ref[...]) * dinv + b1_ref[...], 0.0)
    y2_ref[...] = jnp.dot(h, w2_ref[...],
                          preferred_element_type=jnp.float32) * dinv


def _out_body(s_ref, y2_ref, dinv_ref, b2_ref, o_ref):
    sp = s_ref[...]
    o_ref[...] = ((sp[0, :N] + sp[1, :N] + y2_ref[:N]) * dinv_ref[:N]
                  + b2_ref[...])


_scatter128 = _make_edge_scatter(128, staged=False, tc_tiling=True)
_scatter64 = _make_edge_scatter(64, staged=True, tc_tiling=False)


def kernel(x, edge_index, W1, b1, W2, b2):
    f32 = jnp.float32
    ei = edge_index.astype(jnp.int32)
    pad = N + (jnp.arange(EPAD - E, dtype=jnp.int32) % (NPAD - N))
    src_p = jnp.concatenate([ei[0], pad]).reshape(NW, NCHUNKS, CHUNK)
    dst_p = jnp.concatenate([ei[1], pad]).reshape(NW, NCHUNKS, CHUNK)

    x_pad = jnp.pad(x, ((0, NPAD - N), (0, 0)))
    zeros128 = jnp.zeros((NPAD, 128), f32)
    zeros64 = jnp.zeros((NPAD, 64), f32)
    zeros16 = jnp.zeros((NPAD, 16), f32)
    ones16 = jnp.ones((CHUNK, 16), f32)
    b1r = b1.reshape(1, -1)
    b2r = b2.reshape(1, -1)

    degp = _degree_kernel(dst_p, ones16, zeros16)
    xw1 = pl.pallas_call(
        _mm_body,
        out_shape=jax.ShapeDtypeStruct((NPAD, 128), f32),
    )(x_pad, W1)

    dinv, y1 = pl.pallas_call(
        _dinv_scale_body,
        out_shape=[jax.ShapeDtypeStruct((NPAD, 1), f32),
                   jax.ShapeDtypeStruct((NPAD, 128), f32)],
    )(degp, xw1)

    s1 = _scatter128(y1, src_p, dst_p, zeros128)

    y2 = pl.pallas_call(
        _mid_body,
        out_shape=jax.ShapeDtypeStruct((NPAD, 64), f32),
    )(s1, y1, dinv, b1r, W2)

    s2 = _scatter64(y2, src_p, dst_p, zeros64)

    out = pl.pallas_call(
        _out_body,
        out_shape=jax.ShapeDtypeStruct((N, 64), f32),
    )(s2, y2, dinv, b2r)
    return out


# speedup vs baseline: 1.0376x; 1.0376x over previous
import functools

import jax
import jax.numpy as jnp
from jax import lax
from jax.experimental import pallas as pl
from jax.experimental.pallas import tpu as pltpu
from jax.experimental.pallas import tpu_sc as plsc

N = 10000
E = 320000
NC, NS = 2, 16
NW = NC * NS
CHUNK = 128
EPT = 10240
NCHUNKS = EPT // CHUNK
EPAD = NW * EPT
NPAD = 10240
RPT = NPAD // NS

_MESH = plsc.VectorSubcoreMesh(core_axis_name="c", subcore_axis_name="s")


def _make_edge_scatter(D, staged=False, tc_tiling=True, nbuf=2):
    scratch = [
        pltpu.VMEM((NCHUNKS, CHUNK), jnp.int32),
        [pltpu.VMEM((CHUNK,), jnp.int32) for _ in range(nbuf)],
        [pltpu.VMEM((CHUNK, D), jnp.float32) for _ in range(nbuf)],
        pltpu.VMEM_SHARED((NPAD, D), jnp.float32),
        pltpu.SemaphoreType.DMA((nbuf,)),
        pltpu.SemaphoreType.DMA((nbuf,)),
        pltpu.SemaphoreType.DMA((nbuf,)),
    ]
    if staged:
        scratch.append(pltpu.VMEM_SHARED((NPAD, D), jnp.float32))

    @functools.partial(
        pl.kernel,
        out_type=jax.ShapeDtypeStruct((NC, NPAD, D), jnp.float32),
        mesh=_MESH,
        scratch_types=scratch,
        compiler_params=pltpu.CompilerParams(use_tc_tiling_on_sc=tc_tiling),
    )
    def k(y_hbm, src_hbm, dst_hbm, zeros_hbm, out_hbm,
          didx, sidx, rows, acc, sem_g, sem_i, sem_s, *maybe_table):
        c = lax.axis_index("c")
        s = lax.axis_index("s")
        wid = c * NS + s
        pltpu.sync_copy(zeros_hbm.at[pl.ds(s * RPT, RPT)],
                        acc.at[pl.ds(s * RPT, RPT)])
        if staged:
            table = maybe_table[0]
            pltpu.sync_copy(y_hbm.at[pl.ds(s * RPT, RPT)],
                            table.at[pl.ds(s * RPT, RPT)])
        else:
            table = y_hbm
        pltpu.sync_copy(dst_hbm.at[wid], didx)
        pltpu.sync_copy(src_hbm.at[wid, 0], sidx[0])
        pltpu.sync_copy(src_hbm.at[wid, 1], sidx[1])
        plsc.subcore_barrier()

        def start_sidx(v, b):
            pltpu.async_copy(src_hbm.at[wid, v], sidx[b], sem_i.at[b])

        def wait_sidx(v, b):
            pltpu.make_async_copy(src_hbm.at[wid, v], sidx[b],
                                  sem_i.at[b]).wait()

        def start_gather(b):
            pltpu.async_copy(table.at[sidx[b]], rows[b], sem_g.at[b])

        def wait_gather(b):
            pltpu.make_async_copy(table.at[sidx[b]], rows[b],
                                  sem_g.at[b]).wait()

        def start_scatter(v, b):
            pltpu.async_copy(rows[b], acc.at[didx.at[v]], sem_s.at[b],
                             add=True)

        def wait_scatter(v, b):
            pltpu.make_async_copy(rows[b], acc.at[didx.at[v]],
                                  sem_s.at[b]).wait()

        if nbuf == 2:
            start_gather(0)
            start_gather(1)

            @pl.loop(0, NCHUNKS, step=2)
            def _(kk):
                for b in range(2):
                    v = kk + b
                    wait_gather(b)

                    @pl.when(v + 2 < NCHUNKS)
                    def _():
                        start_sidx(v + 2, b)

                    pltpu.sync_copy(rows[b], acc.at[didx.at[v]], add=True)

                    @pl.when(v + 2 < NCHUNKS)
                    def _():
                        wait_sidx(v + 2, b)
                        start_gather(b)
        else:
            start_sidx(2, 2)
            start_sidx(3, 3)

            @pl.loop(0, NCHUNKS, step=4)
            def _(kk):
                for j in range(4):
                    v = kk + j
                    bs = (j + 2) % 4
                    if j < 2:
                        @pl.when(kk >= 4)
                        def _():
                            wait_gather(bs)
                            start_scatter(v - 2, bs)
                            start_sidx(v + 2, bs)
                    else:
                        wait_gather(bs)
                        start_scatter(v - 2, bs)

                        @pl.when(v + 2 < NCHUNKS)
                        def _():
                            start_sidx(v + 2, bs)

                    @pl.when(kk >= 4)
                    def _():
                        wait_scatter(v - 4, j)

                    if j < 2:
                        @pl.when(kk >= 4)
                        def _():
                            wait_sidx(v, j)
                    else:
                        wait_sidx(v, j)
                    start_gather(j)

            wait_gather(2)
            start_scatter(NCHUNKS - 2, 2)
            wait_gather(3)
            start_scatter(NCHUNKS - 1, 3)
            for b in range(4):
                wait_scatter(NCHUNKS - 4 + b, b)

        plsc.subcore_barrier()
        pltpu.sync_copy(acc.at[pl.ds(s * RPT, RPT)],
                        out_hbm.at[c, pl.ds(s * RPT, RPT)])

    return k


@functools.partial(
    pl.kernel,
    out_type=jax.ShapeDtypeStruct((NC, NPAD, 8), jnp.float32),
    mesh=_MESH,
    scratch_types=[
        pltpu.VMEM((NCHUNKS, CHUNK), jnp.int32),
        pltpu.VMEM((CHUNK, 16), jnp.float32),
        pltpu.VMEM_SHARED((NPAD, 16), jnp.float32),
        pltpu.SemaphoreType.DMA((1,)),
    ],
    compiler_params=pltpu.CompilerParams(use_tc_tiling_on_sc=False),
)
def _degree_kernel(dst_hbm, ones_hbm, zeros_hbm, out_hbm, didx, ones_v, acc,
                   sem):
    c = lax.axis_index("c")
    s = lax.axis_index("s")
    wid = c * NS + s
    pltpu.sync_copy(zeros_hbm.at[pl.ds(s * RPT, RPT)],
                    acc.at[pl.ds(s * RPT, RPT)])
    pltpu.sync_copy(ones_hbm, ones_v)
    pltpu.sync_copy(dst_hbm.at[wid], didx)
    plsc.subcore_barrier()

    @pl.loop(0, NCHUNKS, step=8)
    def _(kk):
        for j in range(8):
            pltpu.async_copy(ones_v, acc.at[didx.at[kk + j]], sem.at[0],
                             add=True)
        for j in range(8):
            pltpu.make_async_copy(ones_v, acc.at[didx.at[kk + j]],
                                  sem.at[0]).wait()

    plsc.subcore_barrier()
    pltpu.sync_copy(acc.at[pl.ds(s * RPT, RPT), pl.ds(0, 8)],
                    out_hbm.at[c, pl.ds(s * RPT, RPT)])


def _mm_body(x_ref, w_ref, o_ref):
    o_ref[...] = jnp.dot(x_ref[...], w_ref[...],
                         preferred_element_type=jnp.float32)


def _dinv_scale_body(degp_ref, xw_ref, dinv_ref, y_ref):
    d = degp_ref[...]
    deg = d[0, :, 0:1] + d[1, :, 0:1] + 1.0
    dinv = lax.rsqrt(deg)
    dinv_ref[...] = dinv
    y_ref[...] = xw_ref[...] * dinv


def _mid_body(s_ref, y1_ref, dinv_ref, b1_ref, w2_ref, y2_ref):
    sp = s_ref[...]
    dinv = dinv_ref[...]
    h = jnp.maximum((sp[0] + sp[1] + y1_ref[...]) * dinv + b1_ref[...], 0.0)
    y2_ref[...] = jnp.dot(h, w2_ref[...],
                          preferred_element_type=jnp.float32) * dinv


def _out_body(s_ref, y2_ref, dinv_ref, b2_ref, o_ref):
    sp = s_ref[...]
    o_ref[...] = ((sp[0, :N] + sp[1, :N] + y2_ref[:N]) * dinv_ref[:N]
                  + b2_ref[...])


_scatter128 = _make_edge_scatter(128, staged=False, tc_tiling=True)
_scatter64 = _make_edge_scatter(64, staged=True, tc_tiling=False, nbuf=4)


def kernel(x, edge_index, W1, b1, W2, b2):
    f32 = jnp.float32
    ei = edge_index.astype(jnp.int32)
    pad = N + jnp.broadcast_to(jnp.arange(NPAD - N, dtype=jnp.int32),
                               ((EPAD - E) // (NPAD - N), NPAD - N)).reshape(-1)
    src_p = jnp.concatenate([ei[0], pad]).reshape(NW, NCHUNKS, CHUNK)
    dst_p = jnp.concatenate([ei[1], pad]).reshape(NW, NCHUNKS, CHUNK)

    x_pad = jnp.pad(x, ((0, NPAD - N), (0, 0)))
    zeros128 = jnp.zeros((NPAD, 128), f32)
    zeros64 = jnp.zeros((NPAD, 64), f32)
    zeros16 = jnp.zeros((NPAD, 16), f32)
    ones16 = jnp.ones((CHUNK, 16), f32)
    b1r = b1.reshape(1, -1)
    b2r = b2.reshape(1, -1)

    degp = _degree_kernel(dst_p, ones16, zeros16)
    xw1 = pl.pallas_call(
        _mm_body,
        out_shape=jax.ShapeDtypeStruct((NPAD, 128), f32),
    )(x_pad, W1)

    dinv, y1 = pl.pallas_call(
        _dinv_scale_body,
        out_shape=[jax.ShapeDtypeStruct((NPAD, 1), f32),
                   jax.ShapeDtypeStruct((NPAD, 128), f32)],
    )(degp, xw1)

    s1 = _scatter128(y1, src_p, dst_p, zeros128)

    y2 = pl.pallas_call(
        _mid_body,
        out_shape=jax.ShapeDtypeStruct((NPAD, 64), f32),
    )(s1, y1, dinv, b1r, W2)

    s2 = _scatter64(y2, src_p, dst_p, zeros64)

    out = pl.pallas_call(
        _out_body,
        out_shape=jax.ShapeDtypeStruct((N, 64), f32),
    )(s2, y2, dinv, b2r)
    return out


# speedup vs baseline: 1.0501x; 1.0120x over previous
import functools

import jax
import jax.numpy as jnp
from jax import lax
from jax.experimental import pallas as pl
from jax.experimental.pallas import tpu as pltpu
from jax.experimental.pallas import tpu_sc as plsc

N = 10000
E = 320000
NC, NS = 2, 16
NW = NC * NS
CHUNK = 128
EPT = 10240
NCHUNKS = EPT // CHUNK
EPAD = NW * EPT
NPAD = 10240
RPT = NPAD // NS

_MESH = plsc.VectorSubcoreMesh(core_axis_name="c", subcore_axis_name="s")


def _make_edge_scatter(D, staged=False, tc_tiling=True, nbuf=2):
    scratch = [
        pltpu.VMEM((NCHUNKS, CHUNK), jnp.int32),
        [pltpu.VMEM((CHUNK,), jnp.int32) for _ in range(nbuf)],
        [pltpu.VMEM((CHUNK, D), jnp.float32) for _ in range(nbuf)],
        pltpu.VMEM_SHARED((NPAD, D), jnp.float32),
        pltpu.SemaphoreType.DMA((nbuf,)),
        pltpu.SemaphoreType.DMA((nbuf,)),
        pltpu.SemaphoreType.DMA((nbuf,)),
    ]
    if staged:
        scratch.append(pltpu.VMEM_SHARED((NPAD, D), jnp.float32))

    @functools.partial(
        pl.kernel,
        out_type=jax.ShapeDtypeStruct((NC, NPAD, D), jnp.float32),
        mesh=_MESH,
        scratch_types=scratch,
        compiler_params=pltpu.CompilerParams(use_tc_tiling_on_sc=tc_tiling),
    )
    def k(y_hbm, src_hbm, dst_hbm, zeros_hbm, out_hbm,
          didx, sidx, rows, acc, sem_g, sem_i, sem_s, *maybe_table):
        c = lax.axis_index("c")
        s = lax.axis_index("s")
        wid = c * NS + s
        pltpu.sync_copy(zeros_hbm.at[pl.ds(s * RPT, RPT)],
                        acc.at[pl.ds(s * RPT, RPT)])
        if staged:
            table = maybe_table[0]
            pltpu.sync_copy(y_hbm.at[pl.ds(s * RPT, RPT)],
                            table.at[pl.ds(s * RPT, RPT)])
        else:
            table = y_hbm
        pltpu.sync_copy(dst_hbm.at[wid], didx)
        pltpu.sync_copy(src_hbm.at[wid, 0], sidx[0])
        pltpu.sync_copy(src_hbm.at[wid, 1], sidx[1])
        plsc.subcore_barrier()

        def start_sidx(v, b):
            pltpu.async_copy(src_hbm.at[wid, v], sidx[b], sem_i.at[b])

        def wait_sidx(v, b):
            pltpu.make_async_copy(src_hbm.at[wid, v], sidx[b],
                                  sem_i.at[b]).wait()

        def start_gather(b):
            pltpu.async_copy(table.at[sidx[b]], rows[b], sem_g.at[b])

        def wait_gather(b):
            pltpu.make_async_copy(table.at[sidx[b]], rows[b],
                                  sem_g.at[b]).wait()

        def start_scatter(v, b):
            pltpu.async_copy(rows[b], acc.at[didx.at[v]], sem_s.at[b],
                             add=True)

        def wait_scatter(v, b):
            pltpu.make_async_copy(rows[b], acc.at[didx.at[v]],
                                  sem_s.at[b]).wait()

        if nbuf == 2:
            start_gather(0)
            start_gather(1)

            @pl.loop(0, NCHUNKS, step=2)
            def _(kk):
                for b in range(2):
                    v = kk + b
                    wait_gather(b)

                    @pl.when(v + 2 < NCHUNKS)
                    def _():
                        start_sidx(v + 2, b)

                    pltpu.sync_copy(rows[b], acc.at[didx.at[v]], add=True)

                    @pl.when(v + 2 < NCHUNKS)
                    def _():
                        wait_sidx(v + 2, b)
                        start_gather(b)
        else:
            start_sidx(2, 2)
            start_sidx(3, 3)

            @pl.loop(0, NCHUNKS, step=4)
            def _(kk):
                for j in range(4):
                    v = kk + j
                    bs = (j + 2) % 4
                    if j < 2:
                        @pl.when(kk >= 4)
                        def _():
                            wait_gather(bs)
                            start_scatter(v - 2, bs)
                            start_sidx(v + 2, bs)
                    else:
                        wait_gather(bs)
                        start_scatter(v - 2, bs)

                        @pl.when(v + 2 < NCHUNKS)
                        def _():
                            start_sidx(v + 2, bs)

                    @pl.when(kk >= 4)
                    def _():
                        wait_scatter(v - 4, j)

                    if j < 2:
                        @pl.when(kk >= 4)
                        def _():
                            wait_sidx(v, j)
                    else:
                        wait_sidx(v, j)
                    start_gather(j)

            wait_gather(2)
            start_scatter(NCHUNKS - 2, 2)
            wait_gather(3)
            start_scatter(NCHUNKS - 1, 3)
            for b in range(4):
                wait_scatter(NCHUNKS - 4 + b, b)

        plsc.subcore_barrier()
        pltpu.sync_copy(acc.at[pl.ds(s * RPT, RPT)],
                        out_hbm.at[c, pl.ds(s * RPT, RPT)])

    return k


@functools.partial(
    pl.kernel,
    out_type=jax.ShapeDtypeStruct((NC, NPAD, 16), jnp.float32),
    mesh=_MESH,
    scratch_types=[
        pltpu.VMEM((NCHUNKS, CHUNK), jnp.int32),
        pltpu.VMEM((CHUNK, 16), jnp.float32),
        pltpu.VMEM_SHARED((NPAD, 16), jnp.float32),
        pltpu.SemaphoreType.DMA((1,)),
    ],
    compiler_params=pltpu.CompilerParams(use_tc_tiling_on_sc=False),
)
def _degree_kernel(dst_hbm, ones_hbm, zeros_hbm, out_hbm, didx, ones_v, acc,
                   sem):
    c = lax.axis_index("c")
    s = lax.axis_index("s")
    wid = c * NS + s
    pltpu.sync_copy(zeros_hbm.at[pl.ds(s * RPT, RPT)],
                    acc.at[pl.ds(s * RPT, RPT)])
    pltpu.sync_copy(ones_hbm, ones_v)
    pltpu.sync_copy(dst_hbm.at[wid], didx)
    plsc.subcore_barrier()

    @pl.loop(0, NCHUNKS, step=8)
    def _(kk):
        for j in range(8):
            pltpu.async_copy(ones_v, acc.at[didx.at[kk + j]], sem.at[0],
                             add=True)
        for j in range(8):
            pltpu.make_async_copy(ones_v, acc.at[didx.at[kk + j]],
                                  sem.at[0]).wait()

    plsc.subcore_barrier()
    pltpu.sync_copy(acc.at[pl.ds(s * RPT, RPT)],
                    out_hbm.at[c, pl.ds(s * RPT, RPT)])


def _mm_body(x_ref, w_ref, o_ref):
    o_ref[...] = jnp.dot(x_ref[...], w_ref[...],
                         preferred_element_type=jnp.float32)


def _dinv_scale_body(degp_ref, xw_ref, dinv_ref, y_ref):
    d = degp_ref[...]
    deg = d[0, :, 0:1] + d[1, :, 0:1] + 1.0
    dinv = lax.rsqrt(deg)
    dinv_ref[...] = dinv
    y_ref[...] = xw_ref[...] * dinv


def _mid_body(s_ref, y1_ref, dinv_ref, b1_ref, w2_ref, y2_ref):
    sp = s_ref[...]
    dinv = dinv_ref[...]
    h = jnp.maximum((sp[0] + sp[1] + y1_ref[...]) * dinv + b1_ref[...], 0.0)
    y2_ref[...] = jnp.dot(h, w2_ref[...],
                          preferred_element_type=jnp.float32) * dinv


def _out_body(s_ref, y2_ref, dinv_ref, b2_ref, o_ref):
    sp = s_ref[...]
    o_ref[...] = (sp[0] + sp[1] + y2_ref[...]) * dinv_ref[...] + b2_ref[...]


_scatter128 = _make_edge_scatter(128, staged=False, tc_tiling=True)
_scatter64 = _make_edge_scatter(64, staged=True, tc_tiling=False, nbuf=4)


def kernel(x, edge_index, W1, b1, W2, b2):
    f32 = jnp.float32
    ei = edge_index.astype(jnp.int32)
    pad = N + jnp.broadcast_to(jnp.arange(NPAD - N, dtype=jnp.int32),
                               ((EPAD - E) // (NPAD - N), NPAD - N)).reshape(-1)
    src_p = jnp.concatenate([ei[0], pad]).reshape(NW, NCHUNKS, CHUNK)
    dst_p = jnp.concatenate([ei[1], pad]).reshape(NW, NCHUNKS, CHUNK)

    x_pad = jnp.pad(x, ((0, NPAD - N), (0, 0)))
    zeros128 = jnp.zeros((NPAD, 128), f32)
    zeros64 = jnp.zeros((NPAD, 64), f32)
    zeros16 = jnp.zeros((NPAD, 16), f32)
    ones16 = jnp.ones((CHUNK, 16), f32)
    b1r = b1.reshape(1, -1)
    b2r = b2.reshape(1, -1)

    degp = _degree_kernel(dst_p, ones16, zeros16)
    xw1 = pl.pallas_call(
        _mm_body,
        out_shape=jax.ShapeDtypeStruct((NPAD, 128), f32),
    )(x_pad, W1)

    BLK = 1280
    NB = NPAD // BLK
    dinv, y1 = pl.pallas_call(
        _dinv_scale_body,
        grid=(NB,),
        in_specs=[pl.BlockSpec((NC, BLK, 16), lambda i: (0, i, 0)),
                  pl.BlockSpec((BLK, 128), lambda i: (i, 0))],
        out_specs=[pl.BlockSpec((BLK, 1), lambda i: (i, 0)),
                   pl.BlockSpec((BLK, 128), lambda i: (i, 0))],
        out_shape=[jax.ShapeDtypeStruct((NPAD, 1), f32),
                   jax.ShapeDtypeStruct((NPAD, 128), f32)],
    )(degp, xw1)

    s1 = _scatter128(y1, src_p, dst_p, zeros128)

    y2 = pl.pallas_call(
        _mid_body,
        grid=(NB,),
        in_specs=[pl.BlockSpec((NC, BLK, 128), lambda i: (0, i, 0)),
                  pl.BlockSpec((BLK, 128), lambda i: (i, 0)),
                  pl.BlockSpec((BLK, 1), lambda i: (i, 0)),
                  pl.BlockSpec((1, 128), lambda i: (0, 0)),
                  pl.BlockSpec((128, 64), lambda i: (0, 0))],
        out_specs=pl.BlockSpec((BLK, 64), lambda i: (i, 0)),
        out_shape=jax.ShapeDtypeStruct((NPAD, 64), f32),
    )(s1, y1, dinv, b1r, W2)

    s2 = _scatter64(y2, src_p, dst_p, zeros64)

    OBLK = 2000
    out = pl.pallas_call(
        _out_body,
        grid=(N // OBLK,),
        in_specs=[pl.BlockSpec((NC, OBLK, 64), lambda i: (0, i, 0)),
                  pl.BlockSpec((OBLK, 64), lambda i: (i, 0)),
                  pl.BlockSpec((OBLK, 1), lambda i: (i, 0)),
                  pl.BlockSpec((1, 64), lambda i: (0, 0))],
        out_specs=pl.BlockSpec((OBLK, 64), lambda i: (i, 0)),
        out_shape=jax.ShapeDtypeStruct((N, 64), f32),
    )(s2, y2, dinv, b2r)
    return out
